# Initial kernel scaffold; baseline (speedup 1.0000x reference)
import functools

import jax
import jax.numpy as jnp
from jax import lax
from jax.experimental import pallas as pl
from jax.experimental.pallas import tpu as pltpu
from jax.experimental.pallas import tpu_sc as plsc

N = 10000
E = 320000
D = 128
D_OUT = 64

NC, NS = 2, 16
NW = NC * NS
E_PER_W = E // NW
CHUNK = 80
N_CHUNKS = E_PER_W // CHUNK
NPAD = 10240
ROWS_PER_TILE = NPAD // NS

_sc_mesh = plsc.VectorSubcoreMesh(core_axis_name="c", subcore_axis_name="s")


@functools.partial(
    pl.kernel,
    out_type=jax.ShapeDtypeStruct((NC, NPAD), jnp.float32),
    mesh=_sc_mesh,
    scratch_types=[
        pltpu.VMEM_SHARED((NPAD,), jnp.float32),
        pltpu.VMEM((ROWS_PER_TILE,), jnp.float32),
        pltpu.VMEM((CHUNK,), jnp.int32),
        pltpu.VMEM((CHUNK,), jnp.float32),
    ],
)
def _deg_kernel(dst_hbm, ew_hbm, out_hbm, acc_sp, zbuf_v, dst_v, ew_v):
    cid = lax.axis_index("c")
    sid = lax.axis_index("s")
    wid = sid * NC + cid
    zeros16 = jnp.zeros((16,), jnp.float32)

    @pl.loop(0, ROWS_PER_TILE // 16)
    def _(i):
        zbuf_v[pl.ds(i * 16, 16)] = zeros16

    pltpu.sync_copy(zbuf_v, acc_sp.at[pl.ds(sid * ROWS_PER_TILE, ROWS_PER_TILE)])
    plsc.subcore_barrier()

    base = wid * E_PER_W

    @pl.loop(0, N_CHUNKS)
    def _(j):
        off = base + j * CHUNK
        pltpu.sync_copy(dst_hbm.at[pl.ds(off, CHUNK)], dst_v)
        pltpu.sync_copy(ew_hbm.at[pl.ds(off, CHUNK)], ew_v)
        pltpu.sync_copy(ew_v, acc_sp.at[dst_v], add=True)

    plsc.subcore_barrier()
    pltpu.sync_copy(
        acc_sp.at[pl.ds(sid * ROWS_PER_TILE, ROWS_PER_TILE)],
        out_hbm.at[cid, pl.ds(sid * ROWS_PER_TILE, ROWS_PER_TILE)],
    )


@functools.partial(
    pl.kernel,
    out_type=jax.ShapeDtypeStruct((NC, NPAD, D), jnp.float32),
    mesh=_sc_mesh,
    scratch_types=[
        pltpu.VMEM_SHARED((NPAD, D), jnp.float32),
        pltpu.VMEM((CHUNK, D), jnp.float32),
        pltpu.VMEM((CHUNK,), jnp.int32),
        pltpu.VMEM((CHUNK,), jnp.int32),
        pltpu.VMEM((CHUNK,), jnp.float32),
        pltpu.SemaphoreType.DMA,
    ],
)
def _agg_kernel(ys_hbm, src_hbm, dst_hbm, ew_hbm, out_hbm,
                acc_sp, rows_v, src_v, dst_v, ew_v, sem):
    cid = lax.axis_index("c")
    sid = lax.axis_index("s")
    wid = sid * NC + cid
    zeros16 = jnp.zeros((16,), jnp.float32)

    @pl.loop(0, CHUNK)
    def _(r):
        for c in range(D // 16):
            rows_v[r, pl.ds(c * 16, 16)] = zeros16

    @pl.loop(0, ROWS_PER_TILE // CHUNK)
    def _(i):
        pltpu.sync_copy(
            rows_v, acc_sp.at[pl.ds(sid * ROWS_PER_TILE + i * CHUNK, CHUNK)])

    plsc.subcore_barrier()
    base = wid * E_PER_W

    @pl.loop(0, N_CHUNKS)
    def _(j):
        off = base + j * CHUNK
        pltpu.sync_copy(src_hbm.at[pl.ds(off, CHUNK)], src_v)
        pltpu.sync_copy(dst_hbm.at[pl.ds(off, CHUNK)], dst_v)
        pltpu.sync_copy(ew_hbm.at[pl.ds(off, CHUNK)], ew_v)
        pltpu.async_copy(ys_hbm.at[src_v], rows_v, sem).wait()

        @pl.loop(0, CHUNK)
        def _(k):
            w = ew_v[k]
            for c in range(D // 16):
                rows_v[k, pl.ds(c * 16, 16)] = rows_v[k, pl.ds(c * 16, 16)] * w

        pltpu.sync_copy(rows_v, acc_sp.at[dst_v], add=True)

    plsc.subcore_barrier()
    pltpu.sync_copy(
        acc_sp.at[pl.ds(sid * ROWS_PER_TILE, ROWS_PER_TILE)],
        out_hbm.at[cid, pl.ds(sid * ROWS_PER_TILE, ROWS_PER_TILE)],
    )


_BT = 1000


def _k1_body(deg_ref, x_ref, w1_ref, ys_ref, dinv_ref):
    deg = deg_ref[0] + deg_ref[1] + 1.0
    dinv = lax.rsqrt(jnp.maximum(deg, 1e-12))
    y = jnp.dot(x_ref[...], w1_ref[...], preferred_element_type=jnp.float32)
    ys_ref[...] = y * dinv
    dinv_ref[...] = dinv


def _k3_body(acc_ref, ys_ref, dinv_ref, b1_ref, hs_ref):
    dinv = dinv_ref[...]
    t = dinv * (acc_ref[0] + acc_ref[1] + ys_ref[...]) + b1_ref[...]
    hs_ref[...] = jnp.maximum(t, 0.0) * dinv


def _k5_body(acc_ref, hs_ref, dinv_ref, w2_ref, b2_ref, w3_ref, b3_ref,
             mean_ref, logstd_ref):
    u = dinv_ref[...] * (acc_ref[0] + acc_ref[1] + hs_ref[...])
    mean_ref[...] = jnp.dot(u, w2_ref[...],
                            preferred_element_type=jnp.float32) + b2_ref[...]
    logstd_ref[...] = jnp.dot(u, w3_ref[...],
                              preferred_element_type=jnp.float32) + b3_ref[...]


def _row_spec(d):
    return pl.BlockSpec((_BT, d), lambda i: (i, 0))


def _pair_spec(d):
    return pl.BlockSpec((2, _BT, d), lambda i: (0, i, 0))


def _full_spec(a, b):
    return pl.BlockSpec((a, b), lambda i: (0, 0))


def kernel(x, edge_index, edge_weight, W1, b1, W2, b2, W3, b3):
    ei = edge_index.astype(jnp.int32)
    src, dst, ew = ei[0], ei[1], edge_weight

    deg_parts = _deg_kernel(dst, ew)
    deg2 = deg_parts[:, :N, None]

    ys, dinv = pl.pallas_call(
        _k1_body,
        grid=(N // _BT,),
        in_specs=[_pair_spec(1), _row_spec(D), _full_spec(D, D)],
        out_specs=[_row_spec(D), _row_spec(1)],
        out_shape=[jax.ShapeDtypeStruct((N, D), jnp.float32),
                   jax.ShapeDtypeStruct((N, 1), jnp.float32)],
    )(deg2, x, W1)

    acc1 = _agg_kernel(ys, src, dst, ew)[:, :N]

    hs = pl.pallas_call(
        _k3_body,
        grid=(N // _BT,),
        in_specs=[_pair_spec(D), _row_spec(D), _row_spec(1), _full_spec(1, D)],
        out_specs=_row_spec(D),
        out_shape=jax.ShapeDtypeStruct((N, D), jnp.float32),
    )(acc1, ys, dinv, b1.reshape(1, D))

    acc2 = _agg_kernel(hs, src, dst, ew)[:, :N]

    mean, logstd = pl.pallas_call(
        _k5_body,
        grid=(N // _BT,),
        in_specs=[_pair_spec(D), _row_spec(D), _row_spec(1),
                  _full_spec(D, D_OUT), _full_spec(1, D_OUT),
                  _full_spec(D, D_OUT), _full_spec(1, D_OUT)],
        out_specs=[_row_spec(D_OUT), _row_spec(D_OUT)],
        out_shape=[jax.ShapeDtypeStruct((N, D_OUT), jnp.float32),
                   jax.ShapeDtypeStruct((N, D_OUT), jnp.float32)],
    )(acc2, hs, dinv, W2, b2.reshape(1, D_OUT), W3, b3.reshape(1, D_OUT))

    return (mean, logstd)


# speedup vs baseline: 11.9020x; 11.9020x over previous
import functools

import jax
import jax.numpy as jnp
from jax import lax
from jax.experimental import pallas as pl
from jax.experimental.pallas import tpu as pltpu
from jax.experimental.pallas import tpu_sc as plsc

N = 10000
E = 320000
D = 128
D_OUT = 64

NC, NS = 2, 16
NW = NC * NS
E_PER_W = E // NW
CHUNK = 80
N_CHUNKS = E_PER_W // CHUNK
NPAD = 10240
ROWS_PER_TILE = NPAD // NS

_sc_mesh = plsc.VectorSubcoreMesh(core_axis_name="c", subcore_axis_name="s")


@functools.partial(
    pl.kernel,
    out_type=jax.ShapeDtypeStruct((NC, NPAD), jnp.float32),
    mesh=_sc_mesh,
    scratch_types=[
        pltpu.VMEM_SHARED((NPAD,), jnp.float32),
        pltpu.VMEM((ROWS_PER_TILE,), jnp.float32),
        pltpu.VMEM((CHUNK,), jnp.int32),
        pltpu.VMEM((CHUNK,), jnp.float32),
    ],
)
def _deg_kernel(dst_hbm, ew_hbm, out_hbm, acc_sp, zbuf_v, dst_v, ew_v):
    cid = lax.axis_index("c")
    sid = lax.axis_index("s")
    wid = sid * NC + cid
    zeros16 = jnp.zeros((16,), jnp.float32)

    @pl.loop(0, ROWS_PER_TILE // 16)
    def _(i):
        zbuf_v[pl.ds(i * 16, 16)] = zeros16

    pltpu.sync_copy(zbuf_v, acc_sp.at[pl.ds(sid * ROWS_PER_TILE, ROWS_PER_TILE)])
    plsc.subcore_barrier()

    base = wid * E_PER_W

    @pl.loop(0, N_CHUNKS)
    def _(j):
        off = base + j * CHUNK
        pltpu.sync_copy(dst_hbm.at[pl.ds(off, CHUNK)], dst_v)
        pltpu.sync_copy(ew_hbm.at[pl.ds(off, CHUNK)], ew_v)
        pltpu.sync_copy(ew_v, acc_sp.at[dst_v], add=True)

    plsc.subcore_barrier()
    pltpu.sync_copy(
        acc_sp.at[pl.ds(sid * ROWS_PER_TILE, ROWS_PER_TILE)],
        out_hbm.at[cid, pl.ds(sid * ROWS_PER_TILE, ROWS_PER_TILE)],
    )


@functools.partial(
    pl.kernel,
    out_type=jax.ShapeDtypeStruct((NC, NPAD, D), jnp.float32),
    mesh=_sc_mesh,
    scratch_types=[
        pltpu.VMEM_SHARED((NPAD, D), jnp.float32),
        pltpu.VMEM((CHUNK, D), jnp.float32),
        pltpu.VMEM((CHUNK,), jnp.int32),
        pltpu.VMEM((CHUNK,), jnp.int32),
        pltpu.VMEM((CHUNK,), jnp.float32),
        pltpu.SemaphoreType.DMA,
    ],
)
def _agg_kernel(ys_hbm, src_hbm, dst_hbm, ew_hbm, out_hbm,
                acc_sp, rows_v, src_v, dst_v, ew_v, sem):
    cid = lax.axis_index("c")
    sid = lax.axis_index("s")
    wid = sid * NC + cid
    zeros16 = jnp.zeros((16,), jnp.float32)

    @pl.loop(0, CHUNK)
    def _(r):
        for c in range(D // 16):
            rows_v[r, pl.ds(c * 16, 16)] = zeros16

    @pl.loop(0, ROWS_PER_TILE // CHUNK)
    def _(i):
        pltpu.sync_copy(
            rows_v, acc_sp.at[pl.ds(sid * ROWS_PER_TILE + i * CHUNK, CHUNK)])

    plsc.subcore_barrier()
    base = wid * E_PER_W

    @pl.loop(0, N_CHUNKS)
    def _(j):
        off = base + j * CHUNK
        pltpu.sync_copy(src_hbm.at[pl.ds(off, CHUNK)], src_v)
        pltpu.sync_copy(dst_hbm.at[pl.ds(off, CHUNK)], dst_v)
        pltpu.sync_copy(ew_hbm.at[pl.ds(off, CHUNK)], ew_v)
        pltpu.async_copy(ys_hbm.at[src_v], rows_v, sem).wait()

        @pl.loop(0, CHUNK // 16)
        def _(g):
            ew16 = ew_v[pl.ds(g * 16, 16)]
            for k16 in range(16):
                w = ew16[k16]
                k = g * 16 + k16
                for c in range(D // 16):
                    rows_v[k, pl.ds(c * 16, 16)] = (
                        rows_v[k, pl.ds(c * 16, 16)] * w)

        pltpu.sync_copy(rows_v, acc_sp.at[dst_v], add=True)

    plsc.subcore_barrier()
    pltpu.sync_copy(
        acc_sp.at[pl.ds(sid * ROWS_PER_TILE, ROWS_PER_TILE)],
        out_hbm.at[cid, pl.ds(sid * ROWS_PER_TILE, ROWS_PER_TILE)],
    )


_BT = 1000


def _k1_body(deg_ref, x_ref, w1_ref, ys_ref, dinv_ref):
    deg = deg_ref[0] + deg_ref[1] + 1.0
    dinv = lax.rsqrt(jnp.maximum(deg, 1e-12))
    y = jnp.dot(x_ref[...], w1_ref[...], preferred_element_type=jnp.float32)
    ys_ref[...] = y * dinv
    dinv_ref[...] = dinv


def _k3_body(acc_ref, ys_ref, dinv_ref, b1_ref, hs_ref):
    dinv = dinv_ref[...]
    t = dinv * (acc_ref[0] + acc_ref[1] + ys_ref[...]) + b1_ref[...]
    hs_ref[...] = jnp.maximum(t, 0.0) * dinv


def _k5_body(acc_ref, hs_ref, dinv_ref, w2_ref, b2_ref, w3_ref, b3_ref,
             mean_ref, logstd_ref):
    u = dinv_ref[...] * (acc_ref[0] + acc_ref[1] + hs_ref[...])
    mean_ref[...] = jnp.dot(u, w2_ref[...],
                            preferred_element_type=jnp.float32) + b2_ref[...]
    logstd_ref[...] = jnp.dot(u, w3_ref[...],
                              preferred_element_type=jnp.float32) + b3_ref[...]


def _row_spec(d):
    return pl.BlockSpec((_BT, d), lambda i: (i, 0))


def _pair_spec(d):
    return pl.BlockSpec((2, _BT, d), lambda i: (0, i, 0))


def _full_spec(a, b):
    return pl.BlockSpec((a, b), lambda i: (0, 0))


def kernel(x, edge_index, edge_weight, W1, b1, W2, b2, W3, b3):
    ei = edge_index.astype(jnp.int32)
    src, dst, ew = ei[0], ei[1], edge_weight

    deg_parts = _deg_kernel(dst, ew)
    deg2 = deg_parts[:, :N, None]

    ys, dinv = pl.pallas_call(
        _k1_body,
        grid=(N // _BT,),
        in_specs=[_pair_spec(1), _row_spec(D), _full_spec(D, D)],
        out_specs=[_row_spec(D), _row_spec(1)],
        out_shape=[jax.ShapeDtypeStruct((N, D), jnp.float32),
                   jax.ShapeDtypeStruct((N, 1), jnp.float32)],
    )(deg2, x, W1)

    acc1 = _agg_kernel(ys, src, dst, ew)[:, :N]

    hs = pl.pallas_call(
        _k3_body,
        grid=(N // _BT,),
        in_specs=[_pair_spec(D), _row_spec(D), _row_spec(1), _full_spec(1, D)],
        out_specs=_row_spec(D),
        out_shape=jax.ShapeDtypeStruct((N, D), jnp.float32),
    )(acc1, ys, dinv, b1.reshape(1, D))

    acc2 = _agg_kernel(hs, src, dst, ew)[:, :N]

    mean, logstd = pl.pallas_call(
        _k5_body,
        grid=(N // _BT,),
        in_specs=[_pair_spec(D), _row_spec(D), _row_spec(1),
                  _full_spec(D, D_OUT), _full_spec(1, D_OUT),
                  _full_spec(D, D_OUT), _full_spec(1, D_OUT)],
        out_specs=[_row_spec(D_OUT), _row_spec(D_OUT)],
        out_shape=[jax.ShapeDtypeStruct((N, D_OUT), jnp.float32),
                   jax.ShapeDtypeStruct((N, D_OUT), jnp.float32)],
    )(acc2, hs, dinv, W2, b2.reshape(1, D_OUT), W3, b3.reshape(1, D_OUT))

    return (mean, logstd)
